# baseline (speedup 1.0000x reference)
import functools

import numpy as np
import jax
import jax.numpy as jnp
from jax import lax
from jax.experimental import pallas as pl
from jax.experimental.pallas import tpu as pltpu
from jax.experimental.pallas import tpu_sc as plsc

B, N, C = 4, 16384, 64
NPOINT, NSAMPLE = 1024, 16
R2 = np.float32(0.2 * 0.2)
D = 128
K = 512
NCHUNK = N // K
QB = 64
QM = 128
H = 64
F_OUT = 128


def _bf(v):
    return v.astype(jnp.bfloat16).astype(jnp.float32)


def _bq_body(xyzt_ref, q_ref, lt_ref, idx_ref):
    q0 = q_ref[0, :, 0:1]
    q1 = q_ref[0, :, 1:2]
    q2 = q_ref[0, :, 2:3]
    qq = (q0 * q0 + q1 * q1) + q2 * q2
    q0b = _bf(q0)
    q1b = _bf(q1)
    q2b = _bf(q2)

    def cond(carry):
        j, _, _, done = carry
        return jnp.logical_and(j < NCHUNK, jnp.logical_not(done))

    def body(carry):
        j, cnt, slots, _ = carry
        n0 = pl.multiple_of(j * K, K)
        x0 = xyzt_ref[0, 0:1, pl.ds(n0, K)]
        x1 = xyzt_ref[0, 1:2, pl.ds(n0, K)]
        x2 = xyzt_ref[0, 2:3, pl.ds(n0, K)]
        xx = (x0 * x0 + x1 * x1) + x2 * x2
        dot = (q0b * _bf(x0) + q1b * _bf(x1)) + q2b * _bf(x2)
        d2 = (qq + xx) - 2.0 * dot
        mask = d2 < R2
        mf = mask.astype(jnp.bfloat16)
        prefix = jnp.dot(mf, lt_ref[...], preferred_element_type=jnp.float32)
        pi = prefix.astype(jnp.int32)
        rank = pi + (cnt - 1)
        pos = lax.broadcasted_iota(jnp.int32, (QB, K), 1) + n0
        cols = []
        for t in range(NSAMPLE):
            hit = jnp.logical_and(mask, rank == t)
            cand = jnp.where(hit, pos, N)
            cols.append(jnp.min(cand, axis=1, keepdims=True))
        slots = jnp.minimum(slots, jnp.concatenate(cols, axis=1))
        cnt = cnt + pi[:, K - 1:K]
        done = jnp.min(cnt) >= NSAMPLE
        return j + 1, cnt, slots, done

    init = (jnp.int32(0),
            jnp.zeros((QB, 1), jnp.int32),
            jnp.full((QB, NSAMPLE), N, jnp.int32),
            jnp.bool_(False))
    _, _, slots, _ = lax.while_loop(cond, body, init)
    first = slots[:, 0:1]
    first = jnp.where(first >= N, 0, first)
    idx_ref[0] = jnp.where(slots >= N, jnp.broadcast_to(first, (QB, NSAMPLE)), slots)


def _mlp_body(g_ref, nx3_ref, w0_ref, b0_ref, w1_ref, b1_ref, w2_ref, b2_ref, out_ref):
    g = (g_ref[0] - nx3_ref[0][None]).reshape(NSAMPLE * QM, D)
    h = jnp.dot(g.astype(jnp.bfloat16), w0_ref[...].astype(jnp.bfloat16),
                preferred_element_type=jnp.float32)
    h = jnp.maximum(h + b0_ref[...], 0.0)
    h = jnp.dot(h.astype(jnp.bfloat16), w1_ref[...].astype(jnp.bfloat16),
                preferred_element_type=jnp.float32)
    h = jnp.maximum(h + b1_ref[...], 0.0)
    h = jnp.dot(h.astype(jnp.bfloat16), w2_ref[...].astype(jnp.bfloat16),
                preferred_element_type=jnp.float32)
    h = jnp.maximum(h + b2_ref[...], 0.0)
    out_ref[0] = jnp.transpose(jnp.max(h.reshape(NSAMPLE, QM, F_OUT), axis=0), (1, 0))


NB = 2048


def _table_body(f_ref, xyz_ref, tab_ref):
    ft = jnp.transpose(f_ref[0], (1, 0))
    z = jnp.zeros((NB, D - 3 - C), jnp.float32)
    tab_ref[0] = jnp.concatenate([xyz_ref[0], ft, z], axis=1)


def _build_table(features, xyz):
    return pl.pallas_call(
        _table_body,
        grid=(B, N // NB),
        in_specs=[pl.BlockSpec((1, C, NB), lambda b, n: (b, 0, n)),
                  pl.BlockSpec((1, NB, 3), lambda b, n: (b, n, 0))],
        out_specs=pl.BlockSpec((1, NB, D), lambda b, n: (b, n, 0)),
        out_shape=jax.ShapeDtypeStruct((B, N, D), jnp.float32),
    )(features, xyz)


def _sc_gather(table_flat, idx_flat, num_rows, chunk):
    info = plsc.get_sparse_core_info()
    nw = info.num_cores * info.num_subcores
    per_w = num_rows // nw
    nchunks = per_w // chunk
    mesh = plsc.VectorSubcoreMesh(core_axis_name="c", subcore_axis_name="s")

    @functools.partial(
        pl.kernel, mesh=mesh,
        out_type=jax.ShapeDtypeStruct((num_rows, D), jnp.float32),
        scratch_types=[pltpu.VMEM((chunk,), jnp.int32),
                       pltpu.VMEM((chunk, D), jnp.float32),
                       pltpu.SemaphoreType.DMA])
    def k(table_hbm, idx_hbm, out_hbm, idx_v, rows_v, sem):
        wid = lax.axis_index("s") * info.num_cores + lax.axis_index("c")
        base = wid * per_w
        for c in range(nchunks):
            off = base + c * chunk
            pltpu.sync_copy(idx_hbm.at[pl.ds(off, chunk)], idx_v)
            pltpu.async_copy(table_hbm.at[idx_v], rows_v, sem).wait()
            pltpu.sync_copy(rows_v, out_hbm.at[pl.ds(off, chunk)])

    return k(table_flat, idx_flat)


def _ball_query(xyzt, nxg, lt):
    return pl.pallas_call(
        _bq_body,
        grid=(B, NPOINT // QB),
        in_specs=[pl.BlockSpec((1, 3, N), lambda b, m: (b, 0, 0)),
                  pl.BlockSpec((1, QB, D), lambda b, m: (b, m, 0)),
                  pl.BlockSpec((K, K), lambda b, m: (0, 0))],
        out_specs=pl.BlockSpec((1, QB, NSAMPLE), lambda b, m: (b, m, 0)),
        out_shape=jax.ShapeDtypeStruct((B, NPOINT, NSAMPLE), jnp.int32),
    )(xyzt, nxg, lt)


def _mlp(g4, nx3, w0t, b0r, w1t, b1r, w2t, b2r):
    return pl.pallas_call(
        _mlp_body,
        grid=(B, NPOINT // QM),
        in_specs=[pl.BlockSpec((1, NSAMPLE, QM, D), lambda b, m: (b, 0, m, 0)),
                  pl.BlockSpec((1, QM, D), lambda b, m: (b, m, 0)),
                  pl.BlockSpec((D, H), lambda b, m: (0, 0)),
                  pl.BlockSpec((1, H), lambda b, m: (0, 0)),
                  pl.BlockSpec((H, H), lambda b, m: (0, 0)),
                  pl.BlockSpec((1, H), lambda b, m: (0, 0)),
                  pl.BlockSpec((H, F_OUT), lambda b, m: (0, 0)),
                  pl.BlockSpec((1, F_OUT), lambda b, m: (0, 0))],
        out_specs=pl.BlockSpec((1, F_OUT, QM), lambda b, m: (b, 0, m)),
        out_shape=jax.ShapeDtypeStruct((B, F_OUT, NPOINT), jnp.float32),
    )(g4, nx3, w0t, b0r, w1t, b1r, w2t, b2r)


def kernel(xyz, features, inds, W0, b0, W1, b1, W2, b2):
    table_flat = _build_table(features, xyz).reshape(B * N, D)
    boff = (jnp.arange(B, dtype=jnp.int32) * N)[:, None]

    inds_flat = (inds + boff).reshape(B * NPOINT)
    nxg = _sc_gather(table_flat, inds_flat, B * NPOINT, 128)
    nxg = nxg.reshape(B, NPOINT, D)
    new_xyz = nxg[:, :, :3]

    xyzt = jnp.transpose(xyz, (0, 2, 1))
    lt = jnp.asarray(np.triu(np.ones((K, K), np.float32)), dtype=jnp.bfloat16)
    idx = _ball_query(xyzt, nxg, lt)

    idx_sm = jnp.transpose(idx, (0, 2, 1))
    idx_flat = (idx_sm + boff[:, :, None]).reshape(B * NPOINT * NSAMPLE)
    g = _sc_gather(table_flat, idx_flat, B * NPOINT * NSAMPLE, 512)
    g4 = g.reshape(B, NSAMPLE, NPOINT, D)

    nx3 = jnp.where(jnp.arange(D) < 3, nxg, 0.0)
    w0t = jnp.pad(W0, ((0, 0), (0, D - 3 - C))).T
    new_features = _mlp(g4, nx3, w0t, b0.reshape(1, H), W1.T, b1.reshape(1, H),
                        W2.T, b2.reshape(1, F_OUT))
    return (new_xyz, new_features, inds)


# speedup vs baseline: 1.2075x; 1.2075x over previous
import functools

import numpy as np
import jax
import jax.numpy as jnp
from jax import lax
from jax.experimental import pallas as pl
from jax.experimental.pallas import tpu as pltpu
from jax.experimental.pallas import tpu_sc as plsc

B, N, C = 4, 16384, 64
NPOINT, NSAMPLE = 1024, 16
R2 = np.float32(0.2 * 0.2)
D = 128
K = 512
NCHUNK = N // K
QB = 64
QM = 128
H = 64
F_OUT = 128


def _bf(v):
    return v.astype(jnp.bfloat16).astype(jnp.float32)


def _bq_body(xyzt_ref, q_ref, lt_ref, idx_ref):
    q0 = q_ref[0, :, 0:1]
    q1 = q_ref[0, :, 1:2]
    q2 = q_ref[0, :, 2:3]
    qq = (q0 * q0 + q1 * q1) + q2 * q2
    q0b = _bf(q0)
    q1b = _bf(q1)
    q2b = _bf(q2)

    def cond(carry):
        j, _, _, done = carry
        return jnp.logical_and(j < NCHUNK, jnp.logical_not(done))

    def body(carry):
        j, cnt, slots, _ = carry
        n0 = pl.multiple_of(j * K, K)
        x0 = xyzt_ref[0, 0:1, pl.ds(n0, K)]
        x1 = xyzt_ref[0, 1:2, pl.ds(n0, K)]
        x2 = xyzt_ref[0, 2:3, pl.ds(n0, K)]
        xx = (x0 * x0 + x1 * x1) + x2 * x2
        dot = (q0b * _bf(x0) + q1b * _bf(x1)) + q2b * _bf(x2)
        d2 = (qq + xx) - 2.0 * dot
        mask = d2 < R2
        mf = mask.astype(jnp.bfloat16)
        prefix = jnp.dot(mf, lt_ref[...], preferred_element_type=jnp.float32)
        mrank = jnp.where(mask, prefix + (cnt - 1.0), -1.0)
        posf = (lax.broadcasted_iota(jnp.int32, (QB, K), 1).astype(jnp.float32)
                + lax.convert_element_type(n0, jnp.float32))
        cols = []
        for t in range(NSAMPLE):
            cand = jnp.where(mrank == jnp.float32(t), posf, jnp.float32(N))
            cols.append(jnp.min(cand, axis=1, keepdims=True))
        slots = jnp.minimum(slots, jnp.concatenate(cols, axis=1))
        cnt = cnt + prefix[:, K - 1:K]
        done = jnp.min(cnt) >= NSAMPLE
        return j + 1, cnt, slots, done

    init = (jnp.int32(0),
            jnp.zeros((QB, 1), jnp.float32),
            jnp.full((QB, NSAMPLE), N, jnp.float32),
            jnp.bool_(False))
    _, _, slotsf, _ = lax.while_loop(cond, body, init)
    slots = slotsf.astype(jnp.int32)
    first = slots[:, 0:1]
    first = jnp.where(first >= N, 0, first)
    idx_ref[0] = jnp.where(slots >= N, jnp.broadcast_to(first, (QB, NSAMPLE)), slots)


def _mlp_body(g_ref, nx3_ref, w0_ref, b0_ref, w1_ref, b1_ref, w2_ref, b2_ref, out_ref):
    g = (g_ref[0] - nx3_ref[0][None]).reshape(NSAMPLE * QM, D)
    h = jnp.dot(g.astype(jnp.bfloat16), w0_ref[...].astype(jnp.bfloat16),
                preferred_element_type=jnp.float32)
    h = jnp.maximum(h + b0_ref[...], 0.0)
    h = jnp.dot(h.astype(jnp.bfloat16), w1_ref[...].astype(jnp.bfloat16),
                preferred_element_type=jnp.float32)
    h = jnp.maximum(h + b1_ref[...], 0.0)
    h = jnp.dot(h.astype(jnp.bfloat16), w2_ref[...].astype(jnp.bfloat16),
                preferred_element_type=jnp.float32)
    h = jnp.maximum(h + b2_ref[...], 0.0)
    out_ref[0] = jnp.transpose(jnp.max(h.reshape(NSAMPLE, QM, F_OUT), axis=0), (1, 0))


NB = 2048


def _table_body(f_ref, xyz_ref, tab_ref):
    ft = jnp.transpose(f_ref[0], (1, 0))
    z = jnp.zeros((NB, D - 3 - C), jnp.float32)
    tab_ref[0] = jnp.concatenate([xyz_ref[0], ft, z], axis=1)


def _build_table(features, xyz):
    return pl.pallas_call(
        _table_body,
        grid=(B, N // NB),
        in_specs=[pl.BlockSpec((1, C, NB), lambda b, n: (b, 0, n)),
                  pl.BlockSpec((1, NB, 3), lambda b, n: (b, n, 0))],
        out_specs=pl.BlockSpec((1, NB, D), lambda b, n: (b, n, 0)),
        out_shape=jax.ShapeDtypeStruct((B, N, D), jnp.float32),
    )(features, xyz)


def _sc_gather(table_flat, idx_flat, num_rows, chunk):
    info = plsc.get_sparse_core_info()
    nw = info.num_cores * info.num_subcores
    per_w = num_rows // nw
    nchunks = per_w // chunk
    mesh = plsc.VectorSubcoreMesh(core_axis_name="c", subcore_axis_name="s")

    @functools.partial(
        pl.kernel, mesh=mesh,
        out_type=jax.ShapeDtypeStruct((num_rows, D), jnp.float32),
        scratch_types=[pltpu.VMEM((chunk,), jnp.int32),
                       pltpu.VMEM((chunk, D), jnp.float32),
                       pltpu.SemaphoreType.DMA])
    def k(table_hbm, idx_hbm, out_hbm, idx_v, rows_v, sem):
        wid = lax.axis_index("s") * info.num_cores + lax.axis_index("c")
        base = wid * per_w
        for c in range(nchunks):
            off = base + c * chunk
            pltpu.sync_copy(idx_hbm.at[pl.ds(off, chunk)], idx_v)
            pltpu.async_copy(table_hbm.at[idx_v], rows_v, sem).wait()
            pltpu.sync_copy(rows_v, out_hbm.at[pl.ds(off, chunk)])

    return k(table_flat, idx_flat)


def _ball_query(xyzt, nxg, lt):
    return pl.pallas_call(
        _bq_body,
        grid=(B, NPOINT // QB),
        in_specs=[pl.BlockSpec((1, 3, N), lambda b, m: (b, 0, 0)),
                  pl.BlockSpec((1, QB, D), lambda b, m: (b, m, 0)),
                  pl.BlockSpec((K, K), lambda b, m: (0, 0))],
        out_specs=pl.BlockSpec((1, QB, NSAMPLE), lambda b, m: (b, m, 0)),
        out_shape=jax.ShapeDtypeStruct((B, NPOINT, NSAMPLE), jnp.int32),
    )(xyzt, nxg, lt)


def _mlp(g4, nx3, w0t, b0r, w1t, b1r, w2t, b2r):
    return pl.pallas_call(
        _mlp_body,
        grid=(B, NPOINT // QM),
        in_specs=[pl.BlockSpec((1, NSAMPLE, QM, D), lambda b, m: (b, 0, m, 0)),
                  pl.BlockSpec((1, QM, D), lambda b, m: (b, m, 0)),
                  pl.BlockSpec((D, H), lambda b, m: (0, 0)),
                  pl.BlockSpec((1, H), lambda b, m: (0, 0)),
                  pl.BlockSpec((H, H), lambda b, m: (0, 0)),
                  pl.BlockSpec((1, H), lambda b, m: (0, 0)),
                  pl.BlockSpec((H, F_OUT), lambda b, m: (0, 0)),
                  pl.BlockSpec((1, F_OUT), lambda b, m: (0, 0))],
        out_specs=pl.BlockSpec((1, F_OUT, QM), lambda b, m: (b, 0, m)),
        out_shape=jax.ShapeDtypeStruct((B, F_OUT, NPOINT), jnp.float32),
    )(g4, nx3, w0t, b0r, w1t, b1r, w2t, b2r)


def kernel(xyz, features, inds, W0, b0, W1, b1, W2, b2):
    table_flat = _build_table(features, xyz).reshape(B * N, D)
    boff = (jnp.arange(B, dtype=jnp.int32) * N)[:, None]

    inds_flat = (inds + boff).reshape(B * NPOINT)
    nxg = _sc_gather(table_flat, inds_flat, B * NPOINT, 128)
    nxg = nxg.reshape(B, NPOINT, D)
    new_xyz = nxg[:, :, :3]

    xyzt = jnp.transpose(xyz, (0, 2, 1))
    lt = jnp.asarray(np.triu(np.ones((K, K), np.float32)), dtype=jnp.bfloat16)
    idx = _ball_query(xyzt, nxg, lt)

    idx_sm = jnp.transpose(idx, (0, 2, 1))
    idx_flat = (idx_sm + boff[:, :, None]).reshape(B * NPOINT * NSAMPLE)
    g = _sc_gather(table_flat, idx_flat, B * NPOINT * NSAMPLE, 512)
    g4 = g.reshape(B, NSAMPLE, NPOINT, D)

    nx3 = jnp.where(jnp.arange(D) < 3, nxg, 0.0)
    w0t = jnp.pad(W0, ((0, 0), (0, D - 3 - C))).T
    new_features = _mlp(g4, nx3, w0t, b0.reshape(1, H), W1.T, b1.reshape(1, H),
                        W2.T, b2.reshape(1, F_OUT))
    return (new_xyz, new_features, inds)


# speedup vs baseline: 1.2696x; 1.0514x over previous
import functools

import numpy as np
import jax
import jax.numpy as jnp
from jax import lax
from jax.experimental import pallas as pl
from jax.experimental.pallas import tpu as pltpu
from jax.experimental.pallas import tpu_sc as plsc

B, N, C = 4, 16384, 64
NPOINT, NSAMPLE = 1024, 16
R2 = np.float32(0.2 * 0.2)
D = 128
K = 512
NCHUNK = N // K
QB = 64
QM = 128
H = 64
F_OUT = 128


def _bf(v):
    return v.astype(jnp.bfloat16).astype(jnp.float32)


def _bq_body(xyzt_ref, q_ref, lt_ref, idx_ref):
    q0 = q_ref[0, :, 0:1]
    q1 = q_ref[0, :, 1:2]
    q2 = q_ref[0, :, 2:3]
    qq = (q0 * q0 + q1 * q1) + q2 * q2
    q0b = _bf(q0)
    q1b = _bf(q1)
    q2b = _bf(q2)

    def cond(carry):
        j, _, _, done = carry
        return jnp.logical_and(j < NCHUNK, jnp.logical_not(done))

    def body(carry):
        j, cnt, slots, _ = carry
        n0 = pl.multiple_of(j * K, K)
        x0 = xyzt_ref[0, 0:1, pl.ds(n0, K)]
        x1 = xyzt_ref[0, 1:2, pl.ds(n0, K)]
        x2 = xyzt_ref[0, 2:3, pl.ds(n0, K)]
        xx = (x0 * x0 + x1 * x1) + x2 * x2
        dot = (q0b * _bf(x0) + q1b * _bf(x1)) + q2b * _bf(x2)
        d2 = (qq + xx) - 2.0 * dot
        mask = d2 < R2
        mf = mask.astype(jnp.bfloat16)
        prefix = jnp.dot(mf, lt_ref[...], preferred_element_type=jnp.float32)
        mrank = jnp.where(mask, prefix + (cnt - 1.0), -1.0)
        posf = (lax.broadcasted_iota(jnp.int32, (QB, K), 1).astype(jnp.float32)
                + lax.convert_element_type(n0, jnp.float32))
        cols = []
        for t in range(NSAMPLE):
            cand = jnp.where(mrank == jnp.float32(t), posf, jnp.float32(N))
            cols.append(jnp.min(cand, axis=1, keepdims=True))
        slots = jnp.minimum(slots, jnp.concatenate(cols, axis=1))
        cnt = cnt + prefix[:, K - 1:K]
        done = jnp.min(cnt) >= NSAMPLE
        return j + 1, cnt, slots, done

    init = (jnp.int32(0),
            jnp.zeros((QB, 1), jnp.float32),
            jnp.full((QB, NSAMPLE), N, jnp.float32),
            jnp.bool_(False))
    _, _, slotsf, _ = lax.while_loop(cond, body, init)
    slots = slotsf.astype(jnp.int32)
    first = slots[:, 0:1]
    first = jnp.where(first >= N, 0, first)
    idx_ref[0] = jnp.where(slots >= N, jnp.broadcast_to(first, (QB, NSAMPLE)), slots)


def _mlp_body(g_ref, nx3_ref, w0_ref, b0_ref, w1_ref, b1_ref, w2_ref, b2_ref, out_ref):
    g = (g_ref[0] - nx3_ref[0][None]).reshape(NSAMPLE * QM, D)
    h = jnp.dot(g.astype(jnp.bfloat16), w0_ref[...].astype(jnp.bfloat16),
                preferred_element_type=jnp.float32)
    h = jnp.maximum(h + b0_ref[...], 0.0)
    h = jnp.dot(h.astype(jnp.bfloat16), w1_ref[...].astype(jnp.bfloat16),
                preferred_element_type=jnp.float32)
    h = jnp.maximum(h + b1_ref[...], 0.0)
    h = jnp.dot(h.astype(jnp.bfloat16), w2_ref[...].astype(jnp.bfloat16),
                preferred_element_type=jnp.float32)
    h = jnp.maximum(h + b2_ref[...], 0.0)
    out_ref[0] = jnp.transpose(jnp.max(h.reshape(NSAMPLE, QM, F_OUT), axis=0), (1, 0))


NB = 2048


def _table_body(f_ref, xyz_ref, tab_ref):
    ft = jnp.transpose(f_ref[0], (1, 0))
    z = jnp.zeros((NB, D - 3 - C), jnp.float32)
    tab_ref[0] = jnp.concatenate([xyz_ref[0], ft, z], axis=1)


def _build_table(features, xyz):
    return pl.pallas_call(
        _table_body,
        grid=(B, N // NB),
        in_specs=[pl.BlockSpec((1, C, NB), lambda b, n: (b, 0, n)),
                  pl.BlockSpec((1, NB, 3), lambda b, n: (b, n, 0))],
        out_specs=pl.BlockSpec((1, NB, D), lambda b, n: (b, n, 0)),
        out_shape=jax.ShapeDtypeStruct((B, N, D), jnp.float32),
    )(features, xyz)


def _sc_gather(table_flat, idx_flat, num_rows, chunk):
    info = plsc.get_sparse_core_info()
    nw = info.num_cores * info.num_subcores
    per_w = num_rows // nw
    nchunks = per_w // chunk
    mesh = plsc.VectorSubcoreMesh(core_axis_name="c", subcore_axis_name="s")

    @functools.partial(
        pl.kernel, mesh=mesh,
        out_type=jax.ShapeDtypeStruct((num_rows, D), jnp.float32),
        scratch_types=[pltpu.VMEM((chunk,), jnp.int32),
                       pltpu.VMEM((chunk, D), jnp.float32),
                       pltpu.SemaphoreType.DMA])
    def k(table_hbm, idx_hbm, out_hbm, idx_v, rows_v, sem):
        wid = lax.axis_index("s") * info.num_cores + lax.axis_index("c")
        base = wid * per_w
        for c in range(nchunks):
            off = base + c * chunk
            pltpu.sync_copy(idx_hbm.at[pl.ds(off, chunk)], idx_v)
            pltpu.async_copy(table_hbm.at[idx_v], rows_v, sem).wait()
            pltpu.sync_copy(rows_v, out_hbm.at[pl.ds(off, chunk)])

    return k(table_flat, idx_flat)


def _ball_query(xyzt, nxg, lt):
    return pl.pallas_call(
        _bq_body,
        grid=(B, NPOINT // QB),
        in_specs=[pl.BlockSpec((1, 3, N), lambda b, m: (b, 0, 0)),
                  pl.BlockSpec((1, QB, D), lambda b, m: (b, m, 0)),
                  pl.BlockSpec((K, K), lambda b, m: (0, 0))],
        out_specs=pl.BlockSpec((1, QB, NSAMPLE), lambda b, m: (b, m, 0)),
        out_shape=jax.ShapeDtypeStruct((B, NPOINT, NSAMPLE), jnp.int32),
    )(xyzt, nxg, lt)


def _mlp(g4, nx3, w0t, b0r, w1t, b1r, w2t, b2r):
    return pl.pallas_call(
        _mlp_body,
        grid=(B, NPOINT // QM),
        in_specs=[pl.BlockSpec((1, NSAMPLE, QM, D), lambda b, m: (b, 0, m, 0)),
                  pl.BlockSpec((1, QM, D), lambda b, m: (b, m, 0)),
                  pl.BlockSpec((D, H), lambda b, m: (0, 0)),
                  pl.BlockSpec((1, H), lambda b, m: (0, 0)),
                  pl.BlockSpec((H, H), lambda b, m: (0, 0)),
                  pl.BlockSpec((1, H), lambda b, m: (0, 0)),
                  pl.BlockSpec((H, F_OUT), lambda b, m: (0, 0)),
                  pl.BlockSpec((1, F_OUT), lambda b, m: (0, 0))],
        out_specs=pl.BlockSpec((1, F_OUT, QM), lambda b, m: (b, 0, m)),
        out_shape=jax.ShapeDtypeStruct((B, F_OUT, NPOINT), jnp.float32),
    )(g4, nx3, w0t, b0r, w1t, b1r, w2t, b2r)


def kernel(xyz, features, inds, W0, b0, W1, b1, W2, b2):
    feats_t = jnp.transpose(features, (0, 2, 1))
    table = jnp.concatenate(
        [xyz, feats_t, jnp.zeros((B, N, D - 3 - C), jnp.float32)], axis=-1)
    table_flat = table.reshape(B * N, D)
    boff = (jnp.arange(B, dtype=jnp.int32) * N)[:, None]

    inds_flat = (inds + boff).reshape(B * NPOINT)
    nxg = _sc_gather(table_flat, inds_flat, B * NPOINT, 128)
    nxg = nxg.reshape(B, NPOINT, D)
    new_xyz = nxg[:, :, :3]

    xyzt = jnp.transpose(xyz, (0, 2, 1))
    lt = jnp.asarray(np.triu(np.ones((K, K), np.float32)), dtype=jnp.bfloat16)
    idx = _ball_query(xyzt, nxg, lt)

    idx_sm = jnp.transpose(idx, (0, 2, 1))
    idx_flat = (idx_sm + boff[:, :, None]).reshape(B * NPOINT * NSAMPLE)
    g = _sc_gather(table_flat, idx_flat, B * NPOINT * NSAMPLE, 512)
    g4 = g.reshape(B, NSAMPLE, NPOINT, D)

    nx3 = jnp.where(jnp.arange(D) < 3, nxg, 0.0)
    w0t = jnp.pad(W0, ((0, 0), (0, D - 3 - C))).T
    new_features = _mlp(g4, nx3, w0t, b0.reshape(1, H), W1.T, b1.reshape(1, H),
                        W2.T, b2.reshape(1, F_OUT))
    return (new_xyz, new_features, inds)


# speedup vs baseline: 1.2702x; 1.0005x over previous
import functools

import numpy as np
import jax
import jax.numpy as jnp
from jax import lax
from jax.experimental import pallas as pl
from jax.experimental.pallas import tpu as pltpu
from jax.experimental.pallas import tpu_sc as plsc

B, N, C = 4, 16384, 64
NPOINT, NSAMPLE = 1024, 16
R2 = np.float32(0.2 * 0.2)
D = 128
K = 1024
NCHUNK = N // K
QB = 64
QM = 128
H = 64
F_OUT = 128


def _bf(v):
    return v.astype(jnp.bfloat16).astype(jnp.float32)


def _bq_body(xyzt_ref, q_ref, lt_ref, idx_ref):
    q0 = q_ref[0, :, 0:1]
    q1 = q_ref[0, :, 1:2]
    q2 = q_ref[0, :, 2:3]
    qq = (q0 * q0 + q1 * q1) + q2 * q2
    q0b = _bf(q0)
    q1b = _bf(q1)
    q2b = _bf(q2)

    def cond(carry):
        j, _, _, done = carry
        return jnp.logical_and(j < NCHUNK, jnp.logical_not(done))

    def body(carry):
        j, cnt, slots, _ = carry
        n0 = pl.multiple_of(j * K, K)
        x0 = xyzt_ref[0, 0:1, pl.ds(n0, K)]
        x1 = xyzt_ref[0, 1:2, pl.ds(n0, K)]
        x2 = xyzt_ref[0, 2:3, pl.ds(n0, K)]
        xx = (x0 * x0 + x1 * x1) + x2 * x2
        dot = (q0b * _bf(x0) + q1b * _bf(x1)) + q2b * _bf(x2)
        d2 = (qq + xx) - 2.0 * dot
        mask = d2 < R2
        mf = mask.astype(jnp.bfloat16)
        prefix = jnp.dot(mf, lt_ref[...], preferred_element_type=jnp.float32)
        mrank = jnp.where(mask, prefix + (cnt - 1.0), -1.0)
        posf = (lax.broadcasted_iota(jnp.int32, (QB, K), 1).astype(jnp.float32)
                + lax.convert_element_type(n0, jnp.float32))
        cols = []
        for t in range(NSAMPLE):
            cand = jnp.where(mrank == jnp.float32(t), posf, jnp.float32(N))
            cols.append(jnp.min(cand, axis=1, keepdims=True))
        slots = jnp.minimum(slots, jnp.concatenate(cols, axis=1))
        cnt = cnt + prefix[:, K - 1:K]
        done = jnp.min(cnt) >= NSAMPLE
        return j + 1, cnt, slots, done

    init = (jnp.int32(0),
            jnp.zeros((QB, 1), jnp.float32),
            jnp.full((QB, NSAMPLE), N, jnp.float32),
            jnp.bool_(False))
    _, _, slotsf, _ = lax.while_loop(cond, body, init)
    slots = slotsf.astype(jnp.int32)
    first = slots[:, 0:1]
    first = jnp.where(first >= N, 0, first)
    idx_ref[0] = jnp.where(slots >= N, jnp.broadcast_to(first, (QB, NSAMPLE)), slots)


def _mlp_body(g_ref, nx3_ref, w0_ref, b0_ref, w1_ref, b1_ref, w2_ref, b2_ref, out_ref):
    g = (g_ref[0] - nx3_ref[0][None]).reshape(NSAMPLE * QM, D)
    h = jnp.dot(g.astype(jnp.bfloat16), w0_ref[...].astype(jnp.bfloat16),
                preferred_element_type=jnp.float32)
    h = jnp.maximum(h + b0_ref[...], 0.0)
    h = jnp.dot(h.astype(jnp.bfloat16), w1_ref[...].astype(jnp.bfloat16),
                preferred_element_type=jnp.float32)
    h = jnp.maximum(h + b1_ref[...], 0.0)
    h = jnp.dot(h.astype(jnp.bfloat16), w2_ref[...].astype(jnp.bfloat16),
                preferred_element_type=jnp.float32)
    h = jnp.maximum(h + b2_ref[...], 0.0)
    out_ref[0] = jnp.transpose(jnp.max(h.reshape(NSAMPLE, QM, F_OUT), axis=0), (1, 0))


def _sc_gather(table_flat, idx_flat, num_rows, chunk):
    info = plsc.get_sparse_core_info()
    nw = info.num_cores * info.num_subcores
    per_w = num_rows // nw
    nchunks = per_w // chunk
    mesh = plsc.VectorSubcoreMesh(core_axis_name="c", subcore_axis_name="s")

    @functools.partial(
        pl.kernel, mesh=mesh,
        out_type=jax.ShapeDtypeStruct((num_rows, D), jnp.float32),
        scratch_types=[pltpu.VMEM((chunk,), jnp.int32),
                       pltpu.VMEM((chunk, D), jnp.float32),
                       pltpu.SemaphoreType.DMA])
    def k(table_hbm, idx_hbm, out_hbm, idx_v, rows_v, sem):
        wid = lax.axis_index("s") * info.num_cores + lax.axis_index("c")
        base = wid * per_w
        for c in range(nchunks):
            off = base + c * chunk
            pltpu.sync_copy(idx_hbm.at[pl.ds(off, chunk)], idx_v)
            pltpu.async_copy(table_hbm.at[idx_v], rows_v, sem).wait()
            pltpu.sync_copy(rows_v, out_hbm.at[pl.ds(off, chunk)])

    return k(table_flat, idx_flat)


def _ball_query(xyzt, nxg, lt):
    return pl.pallas_call(
        _bq_body,
        grid=(B, NPOINT // QB),
        in_specs=[pl.BlockSpec((1, 3, N), lambda b, m: (b, 0, 0)),
                  pl.BlockSpec((1, QB, D), lambda b, m: (b, m, 0)),
                  pl.BlockSpec((K, K), lambda b, m: (0, 0))],
        out_specs=pl.BlockSpec((1, QB, NSAMPLE), lambda b, m: (b, m, 0)),
        out_shape=jax.ShapeDtypeStruct((B, NPOINT, NSAMPLE), jnp.int32),
    )(xyzt, nxg, lt)


def _mlp(g4, nx3, w0t, b0r, w1t, b1r, w2t, b2r):
    return pl.pallas_call(
        _mlp_body,
        grid=(B, NPOINT // QM),
        in_specs=[pl.BlockSpec((1, NSAMPLE, QM, D), lambda b, m: (b, 0, m, 0)),
                  pl.BlockSpec((1, QM, D), lambda b, m: (b, m, 0)),
                  pl.BlockSpec((D, H), lambda b, m: (0, 0)),
                  pl.BlockSpec((1, H), lambda b, m: (0, 0)),
                  pl.BlockSpec((H, H), lambda b, m: (0, 0)),
                  pl.BlockSpec((1, H), lambda b, m: (0, 0)),
                  pl.BlockSpec((H, F_OUT), lambda b, m: (0, 0)),
                  pl.BlockSpec((1, F_OUT), lambda b, m: (0, 0))],
        out_specs=pl.BlockSpec((1, F_OUT, QM), lambda b, m: (b, 0, m)),
        out_shape=jax.ShapeDtypeStruct((B, F_OUT, NPOINT), jnp.float32),
    )(g4, nx3, w0t, b0r, w1t, b1r, w2t, b2r)


def kernel(xyz, features, inds, W0, b0, W1, b1, W2, b2):
    feats_t = jnp.transpose(features, (0, 2, 1))
    table = jnp.concatenate(
        [xyz, feats_t, jnp.zeros((B, N, D - 3 - C), jnp.float32)], axis=-1)
    table_flat = table.reshape(B * N, D)
    boff = (jnp.arange(B, dtype=jnp.int32) * N)[:, None]

    inds_flat = (inds + boff).reshape(B * NPOINT)
    nxg = _sc_gather(table_flat, inds_flat, B * NPOINT, 128)
    nxg = nxg.reshape(B, NPOINT, D)
    new_xyz = nxg[:, :, :3]

    xyzt = jnp.transpose(xyz, (0, 2, 1))
    lt = jnp.asarray(np.triu(np.ones((K, K), np.float32)), dtype=jnp.bfloat16)
    idx = _ball_query(xyzt, nxg, lt)

    idx_sm = jnp.transpose(idx, (0, 2, 1))
    idx_flat = (idx_sm + boff[:, :, None]).reshape(B * NPOINT * NSAMPLE)
    g = _sc_gather(table_flat, idx_flat, B * NPOINT * NSAMPLE, 512)
    g4 = g.reshape(B, NSAMPLE, NPOINT, D)

    nx3 = jnp.where(jnp.arange(D) < 3, nxg, 0.0)
    w0t = jnp.pad(W0, ((0, 0), (0, D - 3 - C))).T
    new_features = _mlp(g4, nx3, w0t, b0.reshape(1, H), W1.T, b1.reshape(1, H),
                        W2.T, b2.reshape(1, F_OUT))
    return (new_xyz, new_features, inds)
